# baseline (speedup 1.0000x reference)
import functools

import jax
import jax.numpy as jnp
from jax import lax
from jax.experimental import pallas as pl
from jax.experimental.pallas import tpu as pltpu
from jax.experimental.pallas import tpu_sc as plsc

_NUM_UNITS = 2048
_D_TOK = 256
_N_TOK = 16384
_N_PAIR = 65536
_H3 = 768
_DEC_H = 1024
_DEC_OUT = 8

_TOK_TILE = 2048
_ROW_TILE = 2048
_GCHUNK = 128
_NW = 32
_CHUNKS = (16384, 16384, 16384, 16384)


def _seg_mean_body(ids_ref, tok_ref, out_ref, acc_ref, cnt_ref):
    step = pl.program_id(0)

    @pl.when(step == 0)
    def _():
        acc_ref[...] = jnp.zeros_like(acc_ref)
        cnt_ref[...] = jnp.zeros_like(cnt_ref)

    ids = ids_ref[...].reshape(1, _TOK_TILE)
    onehot_t = (
        lax.broadcasted_iota(jnp.int32, (_NUM_UNITS, _TOK_TILE), 0) == ids
    ).astype(jnp.bfloat16)
    tok = tok_ref[...].astype(jnp.bfloat16)
    acc_ref[...] += jnp.dot(onehot_t, tok, preferred_element_type=jnp.float32)
    cnt_ref[...] += jnp.dot(onehot_t, jnp.ones((_TOK_TILE, 128), jnp.bfloat16),
                            preferred_element_type=jnp.float32)

    @pl.when(step == pl.num_programs(0) - 1)
    def _():
        out_ref[...] = acc_ref[...] / jnp.maximum(cnt_ref[:, :1], 1.0)


def _sc_gather(table, idx):
    n = idx.shape[0]
    d = table.shape[1]
    per_w = n // _NW
    n_chunks = per_w // _GCHUNK
    mesh = plsc.VectorSubcoreMesh(core_axis_name="c", subcore_axis_name="s")

    @functools.partial(
        pl.kernel,
        out_type=jax.ShapeDtypeStruct((n, d), table.dtype),
        mesh=mesh,
        compiler_params=pltpu.CompilerParams(use_tc_tiling_on_sc=True),
        scratch_types=[
            pltpu.VMEM((per_w,), jnp.int32),
            pltpu.VMEM((_GCHUNK, d), table.dtype),
            pltpu.VMEM((_GCHUNK, d), table.dtype),
            pltpu.VMEM_SHARED((_NUM_UNITS, d), table.dtype),
            pltpu.SemaphoreType.DMA,
            pltpu.SemaphoreType.DMA,
            pltpu.SemaphoreType.DMA,
            pltpu.SemaphoreType.DMA,
        ],
    )
    def gk(table_hbm, idx_hbm, out_hbm, idx_v, row0, row1, tbl_s,
           gsem0, gsem1, osem0, osem1):
        sid = lax.axis_index("s")
        wid = sid * 2 + lax.axis_index("c")
        base = wid * per_w

        @pl.when(sid == 0)
        def _():
            pltpu.sync_copy(table_hbm, tbl_s)

        pltpu.sync_copy(idx_hbm.at[pl.ds(base, per_w)], idx_v)
        plsc.subcore_barrier()

        def g_copy(c, buf, sem):
            return pltpu.make_async_copy(
                tbl_s.at[idx_v.at[pl.ds(c * _GCHUNK, _GCHUNK)]], buf, sem)

        def o_copy(c, buf, sem):
            return pltpu.make_async_copy(
                buf, out_hbm.at[pl.ds(base + c * _GCHUNK, _GCHUNK)], sem)

        g_copy(0, row0, gsem0).start()

        @pl.loop(0, n_chunks // 2)
        def _(i):
            c0 = 2 * i
            c1 = c0 + 1
            g_copy(c0, row0, gsem0).wait()

            @pl.when(i > 0)
            def _():
                o_copy(c1 - 2, row1, osem1).wait()

            g_copy(c1, row1, gsem1).start()
            o_copy(c0, row0, osem0).start()
            g_copy(c1, row1, gsem1).wait()
            o_copy(c0, row0, osem0).wait()

            @pl.when(c1 + 1 < n_chunks)
            def _():
                g_copy(c1 + 1, row0, gsem0).start()

            o_copy(c1, row1, osem1).start()

        o_copy(n_chunks - 1, row1, osem1).wait()

    return gk(table, idx)


def _mlp_body(t0_ref, t1_ref, g1_ref, g2_ref, w1_ref,
              b1_ref, w2_ref, b2_ref, out_ref):
    bf = jnp.bfloat16
    f32 = jnp.float32

    def unpack(gref):
        gi = gref[...]
        lo = pltpu.bitcast(gi << 16, f32).astype(bf)
        hi = pltpu.bitcast(gi & jnp.int32(-65536), f32).astype(bf)
        return lo, hi

    lo1, hi1 = unpack(g1_ref)
    lo2, hi2 = unpack(g2_ref)
    x = jnp.concatenate(
        [t0_ref[...].astype(bf), t1_ref[...].astype(bf),
         lo1, hi1, lo2, hi2], axis=1)
    acc = jnp.dot(x, w1_ref[...], preferred_element_type=f32)
    h = jnp.tanh(acc + b1_ref[...])
    out_ref[...] = jnp.dot(h.astype(bf), w2_ref[...],
                           preferred_element_type=f32) + b2_ref[...]


def kernel(token_embs, segment_ids, p1, p2, tree_pair_embs, W1, b1, W2, b2):
    f32 = jnp.float32
    bf = jnp.bfloat16

    ids3 = segment_ids.astype(jnp.int32).reshape(
        _N_TOK // _TOK_TILE, 1, _TOK_TILE)

    unit_embs = pl.pallas_call(
        _seg_mean_body,
        grid=(_N_TOK // _TOK_TILE,),
        in_specs=[
            pl.BlockSpec((1, 1, _TOK_TILE), lambda i: (i, 0, 0)),
            pl.BlockSpec((_TOK_TILE, _D_TOK), lambda i: (i, 0)),
        ],
        out_specs=pl.BlockSpec((_NUM_UNITS, _D_TOK), lambda i: (0, 0)),
        out_shape=jax.ShapeDtypeStruct((_NUM_UNITS, _D_TOK), f32),
        scratch_shapes=[pltpu.VMEM((_NUM_UNITS, _D_TOK), f32),
                        pltpu.VMEM((_NUM_UNITS, 128), f32)],
    )(ids3, token_embs)

    ub16 = unit_embs.astype(bf)
    upacked = lax.bitcast_convert_type(
        jnp.stack([ub16[:, :_D_TOK // 2], ub16[:, _D_TOK // 2:]], axis=-1),
        jnp.int32)

    w1 = W1.astype(bf)
    b1r = b1.reshape(1, _DEC_H)
    w2 = W2.astype(bf)
    b2r = b2.reshape(1, _DEC_OUT)

    p1i = p1.astype(jnp.int32)
    p2i = p2.astype(jnp.int32)

    outs = []
    base = 0
    for k, chunk in enumerate(_CHUNKS):
        nb = chunk // _ROW_TILE
        row0 = base // _ROW_TILE
        pk = jnp.concatenate([
            lax.dynamic_slice_in_dim(p1i, base, chunk),
            lax.dynamic_slice_in_dim(p2i, base, chunk)])
        base += chunk
        gk = _sc_gather(upacked, pk)
        out_k = pl.pallas_call(
            _mlp_body,
            grid=(nb,),
            in_specs=[
                pl.BlockSpec((_ROW_TILE, _H3 // 2),
                             lambda i, row0=row0: (i + row0, 0)),
                pl.BlockSpec((_ROW_TILE, _H3 // 2),
                             lambda i, row0=row0: (i + row0, 1)),
                pl.BlockSpec((_ROW_TILE, _D_TOK // 2), lambda i: (i, 0)),
                pl.BlockSpec((_ROW_TILE, _D_TOK // 2),
                             lambda i, nb=nb: (i + nb, 0)),
                pl.BlockSpec((_H3 + 2 * _D_TOK, _DEC_H), lambda i: (0, 0)),
                pl.BlockSpec((1, _DEC_H), lambda i: (0, 0)),
                pl.BlockSpec((_DEC_H, _DEC_OUT), lambda i: (0, 0)),
                pl.BlockSpec((1, _DEC_OUT), lambda i: (0, 0)),
            ],
            out_specs=pl.BlockSpec((_ROW_TILE, _DEC_OUT), lambda i: (i, 0)),
            out_shape=jax.ShapeDtypeStruct((chunk, _DEC_OUT), f32),
        )(tree_pair_embs, tree_pair_embs, gk, gk, w1, b1r, w2, b2r)
        outs.append(out_k)
    return jnp.concatenate(outs, axis=0)


# speedup vs baseline: 1.0856x; 1.0856x over previous
import functools

import jax
import jax.numpy as jnp
from jax import lax
from jax.experimental import pallas as pl
from jax.experimental.pallas import tpu as pltpu
from jax.experimental.pallas import tpu_sc as plsc

_NUM_UNITS = 2048
_D_TOK = 256
_N_TOK = 16384
_N_PAIR = 65536
_H3 = 768
_DEC_H = 1024
_DEC_OUT = 8

_TOK_TILE = 2048
_ROW_TILE = 2048
_GCHUNK = 128
_NW = 32
_CHUNKS = (8192, 16384, 20480, 20480)
_UB = 256


def _seg_mean_body(ids_ref, tok_ref, out_ref, acc_ref, cnt_ref):
    step = pl.program_id(0)

    @pl.when(step == 0)
    def _():
        acc_ref[...] = jnp.zeros_like(acc_ref)
        cnt_ref[...] = jnp.zeros_like(cnt_ref)

    ids = ids_ref[...].reshape(1, _TOK_TILE)
    id_lo = jnp.min(ids)
    id_hi = jnp.max(ids)
    tok = tok_ref[...].astype(jnp.bfloat16)
    ones = jnp.ones((_TOK_TILE, 128), jnp.bfloat16)
    for b in range(_NUM_UNITS // _UB):
        lo = b * _UB

        @pl.when((id_hi >= lo) & (id_lo < lo + _UB))
        def _(lo=lo):
            oh = (lax.broadcasted_iota(jnp.int32, (_UB, _TOK_TILE), 0) + lo
                  == ids).astype(jnp.bfloat16)
            acc_ref[lo:lo + _UB, :] += jnp.dot(
                oh, tok, preferred_element_type=jnp.float32)
            cnt_ref[lo:lo + _UB, :] += jnp.dot(
                oh, ones, preferred_element_type=jnp.float32)

    @pl.when(step == pl.num_programs(0) - 1)
    def _():
        out_ref[...] = acc_ref[...] / jnp.maximum(cnt_ref[:, :1], 1.0)


def _sc_gather(table, idx):
    n = idx.shape[0]
    d = table.shape[1]
    per_w = n // _NW
    n_chunks = per_w // _GCHUNK
    mesh = plsc.VectorSubcoreMesh(core_axis_name="c", subcore_axis_name="s")

    @functools.partial(
        pl.kernel,
        out_type=jax.ShapeDtypeStruct((n, d), table.dtype),
        mesh=mesh,
        compiler_params=pltpu.CompilerParams(use_tc_tiling_on_sc=True),
        scratch_types=[
            pltpu.VMEM((per_w,), jnp.int32),
            pltpu.VMEM((_GCHUNK, d), table.dtype),
            pltpu.VMEM((_GCHUNK, d), table.dtype),
            pltpu.VMEM_SHARED((_NUM_UNITS, d), table.dtype),
            pltpu.SemaphoreType.DMA,
            pltpu.SemaphoreType.DMA,
            pltpu.SemaphoreType.DMA,
            pltpu.SemaphoreType.DMA,
        ],
    )
    def gk(table_hbm, idx_hbm, out_hbm, idx_v, row0, row1, tbl_s,
           gsem0, gsem1, osem0, osem1):
        sid = lax.axis_index("s")
        wid = sid * 2 + lax.axis_index("c")
        base = wid * per_w

        @pl.when(sid == 0)
        def _():
            pltpu.sync_copy(table_hbm, tbl_s)

        pltpu.sync_copy(idx_hbm.at[pl.ds(base, per_w)], idx_v)
        plsc.subcore_barrier()

        def g_copy(c, buf, sem):
            return pltpu.make_async_copy(
                tbl_s.at[idx_v.at[pl.ds(c * _GCHUNK, _GCHUNK)]], buf, sem)

        def o_copy(c, buf, sem):
            return pltpu.make_async_copy(
                buf, out_hbm.at[pl.ds(base + c * _GCHUNK, _GCHUNK)], sem)

        g_copy(0, row0, gsem0).start()

        @pl.loop(0, n_chunks // 2)
        def _(i):
            c0 = 2 * i
            c1 = c0 + 1
            g_copy(c0, row0, gsem0).wait()

            @pl.when(i > 0)
            def _():
                o_copy(c1 - 2, row1, osem1).wait()

            g_copy(c1, row1, gsem1).start()
            o_copy(c0, row0, osem0).start()
            g_copy(c1, row1, gsem1).wait()
            o_copy(c0, row0, osem0).wait()

            @pl.when(c1 + 1 < n_chunks)
            def _():
                g_copy(c1 + 1, row0, gsem0).start()

            o_copy(c1, row1, osem1).start()

        o_copy(n_chunks - 1, row1, osem1).wait()

    return gk(table, idx)


def _mlp_body(tree_ref, g1_ref, g2_ref, w1_ref,
              b1_ref, w2_ref, b2_ref, out_ref):
    bf = jnp.bfloat16
    f32 = jnp.float32

    def unpack(gref):
        gi = gref[...]
        lo = pltpu.bitcast(gi << 16, f32).astype(bf)
        hi = pltpu.bitcast(gi & jnp.int32(-65536), f32).astype(bf)
        return lo, hi

    lo1, hi1 = unpack(g1_ref)
    lo2, hi2 = unpack(g2_ref)
    x = jnp.concatenate(
        [tree_ref[...].astype(bf), lo1, hi1, lo2, hi2], axis=1)
    acc = jnp.dot(x, w1_ref[...], preferred_element_type=f32)
    h = jnp.tanh(acc + b1_ref[...])
    out_ref[...] = jnp.dot(h.astype(bf), w2_ref[...],
                           preferred_element_type=f32) + b2_ref[...]


def kernel(token_embs, segment_ids, p1, p2, tree_pair_embs, W1, b1, W2, b2):
    f32 = jnp.float32
    bf = jnp.bfloat16

    ids3 = segment_ids.astype(jnp.int32).reshape(
        _N_TOK // _TOK_TILE, 1, _TOK_TILE)

    unit_embs = pl.pallas_call(
        _seg_mean_body,
        grid=(_N_TOK // _TOK_TILE,),
        in_specs=[
            pl.BlockSpec((1, 1, _TOK_TILE), lambda i: (i, 0, 0)),
            pl.BlockSpec((_TOK_TILE, _D_TOK), lambda i: (i, 0)),
        ],
        out_specs=pl.BlockSpec((_NUM_UNITS, _D_TOK), lambda i: (0, 0)),
        out_shape=jax.ShapeDtypeStruct((_NUM_UNITS, _D_TOK), f32),
        scratch_shapes=[pltpu.VMEM((_NUM_UNITS, _D_TOK), f32),
                        pltpu.VMEM((_NUM_UNITS, 128), f32)],
    )(ids3, token_embs)

    ub16 = unit_embs.astype(bf)
    upacked = lax.bitcast_convert_type(
        jnp.stack([ub16[:, :_D_TOK // 2], ub16[:, _D_TOK // 2:]], axis=-1),
        jnp.int32)

    w1 = W1.astype(bf)
    b1r = b1.reshape(1, _DEC_H)
    w2 = W2.astype(bf)
    b2r = b2.reshape(1, _DEC_OUT)

    p1i = p1.astype(jnp.int32)
    p2i = p2.astype(jnp.int32)

    outs = []
    base = 0
    for k, chunk in enumerate(_CHUNKS):
        nb = chunk // _ROW_TILE
        row0 = base // _ROW_TILE
        pk = jnp.concatenate([
            lax.dynamic_slice_in_dim(p1i, base, chunk),
            lax.dynamic_slice_in_dim(p2i, base, chunk)])
        base += chunk
        gk = _sc_gather(upacked, pk)
        out_k = pl.pallas_call(
            _mlp_body,
            grid=(nb,),
            in_specs=[
                pl.BlockSpec((_ROW_TILE, _H3),
                             lambda i, row0=row0: (i + row0, 0)),
                pl.BlockSpec((_ROW_TILE, _D_TOK // 2), lambda i: (i, 0)),
                pl.BlockSpec((_ROW_TILE, _D_TOK // 2),
                             lambda i, nb=nb: (i + nb, 0)),
                pl.BlockSpec((_H3 + 2 * _D_TOK, _DEC_H), lambda i: (0, 0)),
                pl.BlockSpec((1, _DEC_H), lambda i: (0, 0)),
                pl.BlockSpec((_DEC_H, _DEC_OUT), lambda i: (0, 0)),
                pl.BlockSpec((1, _DEC_OUT), lambda i: (0, 0)),
            ],
            out_specs=pl.BlockSpec((_ROW_TILE, _DEC_OUT), lambda i: (i, 0)),
            out_shape=jax.ShapeDtypeStruct((chunk, _DEC_OUT), f32),
        )(tree_pair_embs, gk, gk, w1, b1r, w2, b2r)
        outs.append(out_k)
    return jnp.concatenate(outs, axis=0)
